# baseline (speedup 1.0000x reference)
import functools

import jax
import jax.numpy as jnp
from jax import lax
from jax.experimental import pallas as pl
from jax.experimental.pallas import tpu as pltpu
from jax.experimental.pallas import tpu_sc as plsc

B = 16384
D = 64
H = 16

NC = 2
NS = 16
NW = NC * NS
BPW = B // NW
G = 64
NCHK = BPW // G


NSLOT = 8


def _sc_gather(table2, idx1):
    mesh = plsc.VectorSubcoreMesh(core_axis_name="c", subcore_axis_name="s")

    @functools.partial(
        pl.kernel,
        out_type=jax.ShapeDtypeStruct((B, D), jnp.float32),
        mesh=mesh,
        scratch_types=[
            pltpu.VMEM((BPW,), jnp.int32),
            pltpu.SMEM((BPW,), jnp.int32),
            pltpu.VMEM((NSLOT, 8, D), jnp.float32),
            pltpu.VMEM((NSLOT, D), jnp.float32),
            [pltpu.SemaphoreType.DMA] * NSLOT,
            pltpu.SemaphoreType.DMA,
        ],
        compiler_params=pltpu.CompilerParams(needs_layout_passes=False),
    )
    def k(table_hbm, idx_hbm, out_hbm, idx_v, idx_s, tile_v, rows_v, sems,
          osem):
        wid = lax.axis_index("s") * NC + lax.axis_index("c")
        base = wid * BPW
        pltpu.sync_copy(idx_hbm.at[pl.ds(base, BPW)], idx_v)
        lanes = lax.iota(jnp.int32, 16)

        def pbody(g, carry):
            v = idx_v[pl.ds(g * 16, 16)]
            for l in range(16):
                idx_s[g * 16 + l] = jnp.max(jnp.where(lanes == l, v, -1))
            return carry

        lax.fori_loop(0, BPW // 16, pbody, 0)

        for s in range(NSLOT):
            c = idx_s[s] >> 3
            pltpu.async_copy(
                table_hbm.at[pl.ds(c * 8, 8)], tile_v.at[s], sems[s]
            )

        def body(g, carry):
            for s in range(NSLOT):
                j = g * NSLOT + s
                pltpu.make_async_copy(
                    table_hbm.at[pl.ds(0, 8)], tile_v.at[s], sems[s]
                ).wait()
                r = idx_s[j] & 7
                for kk in range(D // 16):
                    rows_v[s, pl.ds(kk * 16, 16)] = (
                        tile_v[s, r, pl.ds(kk * 16, 16)]
                    )

                @pl.when(j + NSLOT < BPW)
                def _():
                    c = idx_s[j + NSLOT] >> 3
                    pltpu.async_copy(
                        table_hbm.at[pl.ds(c * 8, 8)], tile_v.at[s], sems[s]
                    )

            pltpu.async_copy(
                rows_v, out_hbm.at[pl.ds(base + g * NSLOT, NSLOT)], osem
            ).wait()
            return carry

        lax.fori_loop(0, BPW // NSLOT, body, 0)

    return k(table2, idx1)


def _mlp_body(x_ref, w1_ref, b1_ref, w2_ref, b2_ref, o_ref):
    x = x_ref[...]
    h = jnp.dot(x, w1_ref[...], preferred_element_type=jnp.float32)
    h = jnp.maximum(h + b1_ref[...], 0.0)
    o_ref[...] = (
        jnp.dot(h, w2_ref[...], preferred_element_type=jnp.float32)
        + b2_ref[...]
    )


def _tc_mlp(embs, W1, b1, W2, b2):
    bm = 2048
    return pl.pallas_call(
        _mlp_body,
        grid=(B // bm,),
        in_specs=[
            pl.BlockSpec((bm, D), lambda i: (i, 0)),
            pl.BlockSpec((D, H), lambda i: (0, 0)),
            pl.BlockSpec((1, H), lambda i: (0, 0)),
            pl.BlockSpec((H, D), lambda i: (0, 0)),
            pl.BlockSpec((1, D), lambda i: (0, 0)),
        ],
        out_specs=pl.BlockSpec((bm, D), lambda i: (i, 0)),
        out_shape=jax.ShapeDtypeStruct((B, D), jnp.float32),
    )(embs, W1, b1.reshape(1, H), W2, b2.reshape(1, D))


def kernel(states, table, W1, b1, W2, b2):
    idx1 = states.reshape(B).astype(jnp.int32)
    embs = _sc_gather(table, idx1)
    sfs = _tc_mlp(embs, W1, b1, W2, b2)
    return (embs, sfs)


# speedup vs baseline: 1.4342x; 1.4342x over previous
import functools

import jax
import jax.numpy as jnp
from jax import lax
from jax.experimental import pallas as pl
from jax.experimental.pallas import tpu as pltpu
from jax.experimental.pallas import tpu_sc as plsc

B = 16384
D = 64
H = 16

NC = 2
NS = 16
NW = NC * NS
BPW = B // NW
NSLOT = 16


def _sc_gather(table3, idx1):
    mesh = plsc.VectorSubcoreMesh(core_axis_name="c", subcore_axis_name="s")

    @functools.partial(
        pl.kernel,
        out_type=jax.ShapeDtypeStruct((B, D), jnp.float32),
        mesh=mesh,
        scratch_types=[
            pltpu.VMEM((BPW,), jnp.int32),
            pltpu.SMEM((BPW,), jnp.int32),
            pltpu.VMEM((NSLOT, 8, D), jnp.float32),
            pltpu.VMEM((BPW, D), jnp.float32),
            [pltpu.SemaphoreType.DMA] * NSLOT,
        ],
        compiler_params=pltpu.CompilerParams(needs_layout_passes=False),
    )
    def k(table_hbm, idx_hbm, out_hbm, idx_v, idx_s, tile_v, rows_v, sems):
        wid = lax.axis_index("s") * NC + lax.axis_index("c")
        base = wid * BPW
        pltpu.sync_copy(idx_hbm.at[pl.ds(base, BPW)], idx_v)
        lanes = lax.iota(jnp.int32, 16)

        def pbody(g, carry):
            v = idx_v[pl.ds(g * 16, 16)]
            for l in range(16):
                idx_s[g * 16 + l] = jnp.max(jnp.where(lanes == l, v, -1))
            return carry

        lax.fori_loop(0, BPW // 16, pbody, 0)

        for s in range(NSLOT):
            c = idx_s[s] >> 3
            pltpu.async_copy(
                table_hbm.at[c], tile_v.at[s], sems[s]
            )

        def body(g, carry):
            for s in range(NSLOT):
                j = g * NSLOT + s
                pltpu.make_async_copy(
                    table_hbm.at[0], tile_v.at[s], sems[s]
                ).wait()
                r = idx_s[j] & 7
                for kk in range(D // 16):
                    rows_v[j, pl.ds(kk * 16, 16)] = (
                        tile_v[s, r, pl.ds(kk * 16, 16)]
                    )

                @pl.when(j + NSLOT < BPW)
                def _():
                    c = idx_s[j + NSLOT] >> 3
                    pltpu.async_copy(
                        table_hbm.at[c], tile_v.at[s], sems[s]
                    )
            return carry

        lax.fori_loop(0, BPW // NSLOT, body, 0)
        pltpu.sync_copy(rows_v, out_hbm.at[pl.ds(base, BPW)])

    return k(table3, idx1)


def _mlp_body(x_ref, w1_ref, b1_ref, w2_ref, b2_ref, oe_ref, os_ref):
    x = x_ref[...]
    h = jnp.dot(x, w1_ref[...], preferred_element_type=jnp.float32)
    h = jnp.maximum(h + b1_ref[...], 0.0)
    y = (
        jnp.dot(h, w2_ref[...], preferred_element_type=jnp.float32)
        + b2_ref[...]
    )
    oe_ref[...] = x.T
    os_ref[...] = y.T


def _tc_mlp(embs, W1, b1, W2, b2):
    bm = 2048
    return pl.pallas_call(
        _mlp_body,
        grid=(B // bm,),
        in_specs=[
            pl.BlockSpec((bm, D), lambda i: (i, 0)),
            pl.BlockSpec((D, H), lambda i: (0, 0)),
            pl.BlockSpec((1, H), lambda i: (0, 0)),
            pl.BlockSpec((H, D), lambda i: (0, 0)),
            pl.BlockSpec((1, D), lambda i: (0, 0)),
        ],
        out_specs=[
            pl.BlockSpec((D, bm), lambda i: (0, i)),
            pl.BlockSpec((D, bm), lambda i: (0, i)),
        ],
        out_shape=[
            jax.ShapeDtypeStruct((D, B), jnp.float32),
            jax.ShapeDtypeStruct((D, B), jnp.float32),
        ],
    )(embs, W1, b1.reshape(1, H), W2, b2.reshape(1, D))


def kernel(states, table, W1, b1, W2, b2):
    idx1 = states.reshape(B).astype(jnp.int32)
    table3 = table.reshape(125000, 8, D)
    embs = _sc_gather(table3, idx1)
    embsT, sfsT = _tc_mlp(embs, W1, b1, W2, b2)
    return (embsT.T, sfsT.T)
